# baseline (speedup 1.0000x reference)
import functools

import numpy as np
import jax
import jax.numpy as jnp
from jax import lax
from jax.experimental import pallas as pl
from jax.experimental.pallas import tpu as pltpu
from jax.experimental.pallas import tpu_sc as plsc

_VOCAB = 1000000
_D = 64
_B = 1024
_L = 200

_NW = 32
_ROWS = _B * _L
_RPW = _ROWS // _NW
_CHUNK = 200
_GSUB = 100
_NCHUNK = _RPW // _CHUNK


def _sinusoidal_pe_np(length, d_model):
    pos = np.arange(length, dtype=np.float32)[:, None]
    div = np.exp(
        np.arange(0, d_model, 2, dtype=np.float32) * (-np.log(10000.0) / d_model)
    )
    pe = np.zeros((length, d_model), dtype=np.float32)
    pe[:, 0::2] = np.sin(pos * div)
    pe[:, 1::2] = np.cos(pos * div)
    return pe


_mesh = plsc.VectorSubcoreMesh(core_axis_name="c", subcore_axis_name="s")


@functools.partial(
    pl.kernel,
    mesh=_mesh,
    compiler_params=pltpu.CompilerParams(use_tc_tiling_on_sc=False),
    out_type=jax.ShapeDtypeStruct((_ROWS, _D), jnp.float32),
    scratch_types=[
        pltpu.VMEM((2 * _NCHUNK, _GSUB), jnp.int32),
        pltpu.VMEM((_L, _D), jnp.float32),
        pltpu.VMEM((2, _CHUNK, _D), jnp.float32),
        pltpu.VMEM((2, _CHUNK, _D), jnp.float32),
        pltpu.SemaphoreType.DMA,
        pltpu.SemaphoreType.DMA,
        pltpu.SemaphoreType.DMA,
        pltpu.SemaphoreType.DMA,
    ],
)
def _embed_kernel(seq_hbm, pe_hbm, table_hbm, out_hbm,
                  idx_v, pe_v, inb, outb, g0, g1, s0, s1):
    wid = lax.axis_index("s") * 2 + lax.axis_index("c")
    base = wid * _RPW
    gsem = (g0, g1)
    ssem = (s0, s1)

    pltpu.sync_copy(seq_hbm.at[wid], idx_v)
    pltpu.sync_copy(pe_hbm, pe_v)

    def gather(i, b):
        pltpu.async_copy(table_hbm.at[idx_v.at[2 * i]],
                         inb.at[b, pl.ds(0, _GSUB)], gsem[b])
        pltpu.async_copy(table_hbm.at[idx_v.at[2 * i + 1]],
                         inb.at[b, pl.ds(_GSUB, _GSUB)], gsem[b])

    def wait_gather(b):
        for _ in range(2):
            pltpu.make_async_copy(table_hbm.at[idx_v.at[0]],
                                  inb.at[b, pl.ds(0, _GSUB)], gsem[b]).wait()

    def store(i, b):
        pltpu.async_copy(outb.at[b],
                         out_hbm.at[pl.ds(base + i * _CHUNK, _CHUNK)], ssem[b])

    def wait_store(b):
        pltpu.make_async_copy(
            outb.at[b], out_hbm.at[pl.ds(base, _CHUNK)], ssem[b]).wait()

    def add_pe(b):
        def row(r, _):
            for d in range(_D // 16):
                sl = pl.ds(d * 16, 16)
                outb[b, r, sl] = inb[b, r, sl] + pe_v[r, sl]
            return 0

        lax.fori_loop(0, _CHUNK, row, 0)

    gather(0, 0)
    gather(1, 1)
    for b in (0, 1):
        wait_gather(b)
        add_pe(b)
        store(b, b)
        gather(b + 2, b)

    def body(i2, _):
        for b in (0, 1):
            i = 2 * i2 + b
            wait_gather(b)
            wait_store(b)
            add_pe(b)
            store(i, b)
            gather(i + 2, b)
        return 0

    lax.fori_loop(1, _NCHUNK // 2 - 1, body, 0)

    for b in (0, 1):
        i = _NCHUNK - 2 + b
        wait_gather(b)
        wait_store(b)
        add_pe(b)
        store(i, b)
    wait_store(0)
    wait_store(1)


def kernel(sequence, token_table):
    seq = sequence.reshape(-1).astype(jnp.int32).reshape(_NW, 2 * _NCHUNK, _GSUB)
    pe = jnp.asarray(_sinusoidal_pe_np(_L, _D))
    out = _embed_kernel(seq, pe, token_table)
    return out.reshape(_B, _L, _D)


# speedup vs baseline: 1.0002x; 1.0002x over previous
import functools

import numpy as np
import jax
import jax.numpy as jnp
from jax import lax
from jax.experimental import pallas as pl
from jax.experimental.pallas import tpu as pltpu
from jax.experimental.pallas import tpu_sc as plsc

_VOCAB = 1000000
_D = 64
_B = 1024
_L = 200

_NW = 32
_ROWS = _B * _L
_RPW = _ROWS // _NW
_CHUNK = 200
_GSUB = 100
_NCHUNK = _RPW // _CHUNK


def _sinusoidal_pe_np(length, d_model):
    pos = np.arange(length, dtype=np.float32)[:, None]
    div = np.exp(
        np.arange(0, d_model, 2, dtype=np.float32) * (-np.log(10000.0) / d_model)
    )
    pe = np.zeros((length, d_model), dtype=np.float32)
    pe[:, 0::2] = np.sin(pos * div)
    pe[:, 1::2] = np.cos(pos * div)
    return pe


_mesh = plsc.VectorSubcoreMesh(core_axis_name="c", subcore_axis_name="s")


@functools.partial(
    pl.kernel,
    mesh=_mesh,
    compiler_params=pltpu.CompilerParams(use_tc_tiling_on_sc=False),
    out_type=jax.ShapeDtypeStruct((_B, _L, _D), jnp.float32),
    scratch_types=[
        pltpu.VMEM((2 * _NCHUNK, _GSUB), jnp.int32),
        pltpu.VMEM((_L, _D), jnp.float32),
        pltpu.VMEM((2, _CHUNK, _D), jnp.float32),
        pltpu.VMEM((2, _CHUNK, _D), jnp.float32),
        pltpu.SemaphoreType.DMA,
        pltpu.SemaphoreType.DMA,
        pltpu.SemaphoreType.DMA,
        pltpu.SemaphoreType.DMA,
    ],
)
def _embed_kernel(seq_hbm, pe_hbm, table_hbm, out_hbm,
                  idx_v, pe_v, inb, outb, g0, g1, s0, s1):
    wid = lax.axis_index("s") * 2 + lax.axis_index("c")
    bbase = wid * (_RPW // _L)
    gsem = (g0, g1)
    ssem = (s0, s1)

    pltpu.sync_copy(seq_hbm.at[wid], idx_v)
    pltpu.sync_copy(pe_hbm, pe_v)

    def gather(i, b):
        pltpu.async_copy(table_hbm.at[idx_v.at[2 * i]],
                         inb.at[b, pl.ds(0, _GSUB)], gsem[b])
        pltpu.async_copy(table_hbm.at[idx_v.at[2 * i + 1]],
                         inb.at[b, pl.ds(_GSUB, _GSUB)], gsem[b])

    def wait_gather(b):
        for _ in range(2):
            pltpu.make_async_copy(table_hbm.at[idx_v.at[0]],
                                  inb.at[b, pl.ds(0, _GSUB)], gsem[b]).wait()

    def store(i, b):
        pltpu.async_copy(outb.at[b], out_hbm.at[bbase + i], ssem[b])

    def wait_store(b):
        pltpu.make_async_copy(outb.at[b], out_hbm.at[bbase], ssem[b]).wait()

    def add_pe(b):
        def row(r, _):
            for d in range(_D // 16):
                sl = pl.ds(d * 16, 16)
                outb[b, r, sl] = inb[b, r, sl] + pe_v[r, sl]
            return 0

        lax.fori_loop(0, _CHUNK, row, 0)

    gather(0, 0)
    gather(1, 1)
    for b in (0, 1):
        wait_gather(b)
        add_pe(b)
        store(b, b)
        gather(b + 2, b)

    def body(i2, _):
        for b in (0, 1):
            i = 2 * i2 + b
            wait_gather(b)
            wait_store(b)
            add_pe(b)
            store(i, b)
            gather(i + 2, b)
        return 0

    lax.fori_loop(1, _NCHUNK // 2 - 1, body, 0)

    for b in (0, 1):
        i = _NCHUNK - 2 + b
        wait_gather(b)
        wait_store(b)
        add_pe(b)
        store(i, b)
    wait_store(0)
    wait_store(1)


def kernel(sequence, token_table):
    seq = sequence.reshape(-1).astype(jnp.int32).reshape(_NW, 2 * _NCHUNK, _GSUB)
    pe = jnp.asarray(_sinusoidal_pe_np(_L, _D))
    return _embed_kernel(seq, pe, token_table)
